# baseline (speedup 1.0000x reference)
import jax
import jax.numpy as jnp
from jax import lax
from jax.experimental import pallas as pl
from jax.experimental.pallas import tpu as pltpu
from jax.experimental.pallas import tpu_sc as plsc

N_NODES = 10000
N_EDGES = 10000
NNZ = 320000
D = 128

CHUNK = 128
NC = 2
NS = 16
NW = NC * NS
NPAD = 10240
PAD_IDX = 10000
ROWS_T = NPAD // NS
STRIP = 32

NCHUNKS_P = 2560
NNZ_P = NCHUNKS_P * CHUNK
CPT = NCHUNKS_P // NW
CPT_D = NCHUNKS_P // NS
KB = 8
NBLK = CPT // KB
NBLK_D = CPT_D // KB

_BLK = 2048
_OBLK = 2000


def _project_body(y_ref, o_ref):
    y = y_ref[...]
    ss = jnp.sum(y * y, axis=1, keepdims=True)
    o_ref[...] = y / jnp.maximum(jnp.sqrt(ss), 1e-7)


def _combine_body(p_ref, o_ref):
    o_ref[...] = p_ref[0] + p_ref[1]


def _final_body(p_ref, th_ref, b_ref, o_ref):
    h = p_ref[0] + p_ref[1]
    acc = jnp.dot(h, th_ref[...], preferred_element_type=jnp.float32)
    o_ref[...] = jnp.tanh(acc + b_ref[...])


def _project(y_pad):
    return pl.pallas_call(
        _project_body,
        grid=(NPAD // _BLK,),
        in_specs=[pl.BlockSpec((_BLK, D), lambda i: (i, 0))],
        out_specs=pl.BlockSpec((_BLK, D), lambda i: (i, 0)),
        out_shape=jax.ShapeDtypeStruct((NPAD, D), jnp.float32),
    )(y_pad)


def _combine(parts):
    return pl.pallas_call(
        _combine_body,
        grid=(NPAD // _BLK,),
        in_specs=[pl.BlockSpec((NC, _BLK, D), lambda i: (0, i, 0))],
        out_specs=pl.BlockSpec((_BLK, D), lambda i: (i, 0)),
        out_shape=jax.ShapeDtypeStruct((NPAD, D), jnp.float32),
    )(parts)


def _final(parts, theta, bias):
    return pl.pallas_call(
        _final_body,
        grid=(N_NODES // _OBLK,),
        in_specs=[
            pl.BlockSpec((NC, _OBLK, D), lambda i: (0, i, 0)),
            pl.BlockSpec((D, D), lambda i: (0, 0)),
            pl.BlockSpec((1, D), lambda i: (0, 0)),
        ],
        out_specs=pl.BlockSpec((_OBLK, D), lambda i: (i, 0)),
        out_shape=jax.ShapeDtypeStruct((N_NODES, D), jnp.float32),
    )(parts, theta, bias.reshape(1, D))


def _sc_mesh():
    return plsc.VectorSubcoreMesh(core_axis_name="c", subcore_axis_name="s")


def _make_agg_body(gather_row, scatter_row):
    def body(table_hbm, inc3_hbm, zeros2d_hbm, zdeg_hbm, ones_hbm,
             parts_hbm,
             gidx2, sidx2, rows0, rows1, ones_v, acc_v, degl_v,
             semg0, semg1, semd,
             acc_sh, deg_sh):
        cid = lax.axis_index("c")
        sid = lax.axis_index("s")
        wid = sid * NC + cid
        r0 = sid * ROWS_T

        pltpu.sync_copy(ones_hbm, ones_v)

        pltpu.sync_copy(zeros2d_hbm, acc_sh.at[pl.ds(r0, ROWS_T)])
        pltpu.sync_copy(zdeg_hbm, deg_sh.at[pl.ds(r0, ROWS_T)])

        plsc.subcore_barrier()

        c0 = wid * CPT
        rows = (rows0, rows1)
        sems = (semg0, semg1)

        def feat_body(bi, carry):
            cb = c0 + bi * KB
            pltpu.sync_copy(inc3_hbm.at[gather_row, pl.ds(cb, KB)], gidx2)
            pltpu.sync_copy(inc3_hbm.at[scatter_row, pl.ds(cb, KB)], sidx2)
            for j in range(KB):
                pltpu.sync_copy(table_hbm.at[gidx2.at[j]], rows[j % 2])
                pltpu.sync_copy(rows[j % 2], acc_sh.at[sidx2.at[j]], add=True)
            return carry

        lax.fori_loop(0, NBLK, feat_body, 0)

        c0d = sid * CPT_D

        def deg_body(bi, carry):
            cb = c0d + bi * KB
            pltpu.sync_copy(inc3_hbm.at[scatter_row, pl.ds(cb, KB)], sidx2)
            pend = []
            for j in range(KB):
                pend.append(pltpu.async_copy(
                    ones_v, deg_sh.at[sidx2.at[j]], semd, add=True))
            for p in pend:
                p.wait()
            return carry

        lax.fori_loop(0, NBLK_D, deg_body, 0)

        plsc.subcore_barrier()

        def emit_body(si, carry):
            sr0 = r0 + si * STRIP
            pltpu.sync_copy(acc_sh.at[pl.ds(sr0, STRIP)], acc_v)
            pltpu.sync_copy(deg_sh.at[pl.ds(sr0, STRIP)], degl_v)
            for g in range(STRIP // 16):
                dv = degl_v[pl.ds(16 * g, 16)]
                rec = 1.0 / jnp.maximum(dv, 1.0)
                for k in range(16):
                    r = 16 * g + k
                    s = rec[k]
                    for j in range(D // 16):
                        acc_v[r, pl.ds(16 * j, 16)] = acc_v[r, pl.ds(16 * j, 16)] * s
            pltpu.sync_copy(acc_v, parts_hbm.at[cid, pl.ds(sr0, STRIP)])
            return carry

        lax.fori_loop(0, ROWS_T // STRIP, emit_body, 0)

    return body


def _aggregate(table, inc3, gather_row, scatter_row, zeros2d, zdeg, ones):
    return pl.kernel(
        _make_agg_body(gather_row, scatter_row),
        out_type=jax.ShapeDtypeStruct((NC, NPAD, D), jnp.float32),
        mesh=_sc_mesh(),
        scratch_types=[
            pltpu.VMEM((KB, CHUNK), jnp.int32),
            pltpu.VMEM((KB, CHUNK), jnp.int32),
            pltpu.VMEM((CHUNK, D), jnp.float32),
            pltpu.VMEM((CHUNK, D), jnp.float32),
            pltpu.VMEM((CHUNK,), jnp.float32),
            pltpu.VMEM((STRIP, D), jnp.float32),
            pltpu.VMEM((STRIP,), jnp.float32),
            pltpu.SemaphoreType.DMA,
            pltpu.SemaphoreType.DMA,
            pltpu.SemaphoreType.DMA,
            pltpu.VMEM_SHARED((NPAD, D), jnp.float32),
            pltpu.VMEM_SHARED((NPAD,), jnp.float32),
        ],
    )(table, inc3, zeros2d, zdeg, ones)


@jax.jit
def kernel(t, y, incidence, theta, bias):
    del t
    zeros2d = jnp.zeros((ROWS_T, D), jnp.float32)
    zdeg = jnp.zeros((ROWS_T,), jnp.float32)
    ones = jnp.ones((CHUNK,), jnp.float32)

    y_pad = jnp.pad(y, ((0, NPAD - N_NODES), (0, 0)))
    inc3 = jnp.pad(incidence, ((0, 0), (0, NNZ_P - NNZ)),
                   constant_values=PAD_IDX).reshape(2, NCHUNKS_P, CHUNK)

    y_proj = _project(y_pad)
    edge_parts = _aggregate(y_proj, inc3, 0, 1, zeros2d, zdeg, ones)
    edge_feat = _combine(edge_parts)
    node_parts = _aggregate(edge_feat, inc3, 1, 0, zeros2d, zdeg, ones)
    return _final(node_parts, theta, bias)


# speedup vs baseline: 2.6878x; 2.6878x over previous
import jax
import jax.numpy as jnp
from jax import lax
from jax.experimental import pallas as pl
from jax.experimental.pallas import tpu as pltpu
from jax.experimental.pallas import tpu_sc as plsc

N_NODES = 10000
N_EDGES = 10000
NNZ = 320000
D = 128

CHUNK = 128
NC = 2
NS = 16
NW = NC * NS
NPAD = 10240
PAD_IDX = 10000
ROWS_T = NPAD // NS
STRIP = 32

NCHUNKS_P = 2560
NNZ_P = NCHUNKS_P * CHUNK
CPT = NCHUNKS_P // NW
CPT_D = NCHUNKS_P // NS
KB = 8
NBLK = CPT // KB
NBLK_D = CPT_D // KB

_BLK = 2048
_OBLK = 2000


def _project_body(y_ref, o_ref):
    y = y_ref[...]
    ss = jnp.sum(y * y, axis=1, keepdims=True)
    o_ref[...] = y / jnp.maximum(jnp.sqrt(ss), 1e-7)


def _combine_body(p_ref, o_ref):
    o_ref[...] = p_ref[0] + p_ref[1]


def _final_body(p_ref, th_ref, b_ref, o_ref):
    h = p_ref[0] + p_ref[1]
    acc = jnp.dot(h, th_ref[...], preferred_element_type=jnp.float32)
    o_ref[...] = jnp.tanh(acc + b_ref[...])


def _project(y_pad):
    return pl.pallas_call(
        _project_body,
        grid=(NPAD // _BLK,),
        in_specs=[pl.BlockSpec((_BLK, D), lambda i: (i, 0))],
        out_specs=pl.BlockSpec((_BLK, D), lambda i: (i, 0)),
        out_shape=jax.ShapeDtypeStruct((NPAD, D), jnp.float32),
    )(y_pad)


def _combine(parts):
    return pl.pallas_call(
        _combine_body,
        grid=(NPAD // _BLK,),
        in_specs=[pl.BlockSpec((NC, _BLK, D), lambda i: (0, i, 0))],
        out_specs=pl.BlockSpec((_BLK, D), lambda i: (i, 0)),
        out_shape=jax.ShapeDtypeStruct((NPAD, D), jnp.float32),
    )(parts)


def _final(parts, theta, bias):
    return pl.pallas_call(
        _final_body,
        grid=(N_NODES // _OBLK,),
        in_specs=[
            pl.BlockSpec((NC, _OBLK, D), lambda i: (0, i, 0)),
            pl.BlockSpec((D, D), lambda i: (0, 0)),
            pl.BlockSpec((1, D), lambda i: (0, 0)),
        ],
        out_specs=pl.BlockSpec((_OBLK, D), lambda i: (i, 0)),
        out_shape=jax.ShapeDtypeStruct((N_NODES, D), jnp.float32),
    )(parts, theta, bias.reshape(1, D))


def _sc_mesh():
    return plsc.VectorSubcoreMesh(core_axis_name="c", subcore_axis_name="s")


def _make_agg_body(gather_row, scatter_row):
    def body(table_hbm, inc3_hbm, zeros2d_hbm, zdeg_hbm, ones_hbm,
             parts_hbm,
             gidx2, sidx2, rows0, rows1, ones_v, acc_v, degl_v,
             semg0, semg1, semd,
             acc_sh, deg_sh):
        cid = lax.axis_index("c")
        sid = lax.axis_index("s")
        wid = sid * NC + cid
        r0 = sid * ROWS_T

        pltpu.sync_copy(ones_hbm, ones_v)

        pltpu.sync_copy(zeros2d_hbm, acc_sh.at[pl.ds(r0, ROWS_T)])
        pltpu.sync_copy(zdeg_hbm, deg_sh.at[pl.ds(r0, ROWS_T)])

        plsc.subcore_barrier()

        c0 = wid * CPT
        rows = (rows0, rows1)
        sems = (semg0, semg1)

        def feat_body(bi, carry):
            cb = c0 + bi * KB
            pltpu.sync_copy(inc3_hbm.at[gather_row, pl.ds(cb, KB)], gidx2)
            pltpu.sync_copy(inc3_hbm.at[scatter_row, pl.ds(cb, KB)], sidx2)
            pend = pltpu.async_copy(
                table_hbm.at[gidx2.at[0]], rows0, semg0)
            for j in range(KB):
                pend.wait()
                if j + 1 < KB:
                    pend = pltpu.async_copy(
                        table_hbm.at[gidx2.at[j + 1]],
                        rows[(j + 1) % 2], sems[(j + 1) % 2])
                pltpu.sync_copy(rows[j % 2], acc_sh.at[sidx2.at[j]], add=True)
            return carry

        lax.fori_loop(0, NBLK, feat_body, 0)

        c0d = sid * CPT_D

        def deg_body(bi, carry):
            cb = c0d + bi * KB
            pltpu.sync_copy(inc3_hbm.at[scatter_row, pl.ds(cb, KB)], sidx2)
            pend = []
            for j in range(KB):
                pend.append(pltpu.async_copy(
                    ones_v, deg_sh.at[sidx2.at[j]], semd, add=True))
            for p in pend:
                p.wait()
            return carry

        lax.fori_loop(0, NBLK_D, deg_body, 0)

        plsc.subcore_barrier()

        def emit_body(si, carry):
            sr0 = r0 + si * STRIP
            pltpu.sync_copy(acc_sh.at[pl.ds(sr0, STRIP)], acc_v)
            pltpu.sync_copy(deg_sh.at[pl.ds(sr0, STRIP)], degl_v)
            for g in range(STRIP // 16):
                dv = degl_v[pl.ds(16 * g, 16)]
                rec = 1.0 / jnp.maximum(dv, 1.0)
                for k in range(16):
                    r = 16 * g + k
                    s = rec[k]
                    for j in range(D // 16):
                        acc_v[r, pl.ds(16 * j, 16)] = acc_v[r, pl.ds(16 * j, 16)] * s
            pltpu.sync_copy(acc_v, parts_hbm.at[cid, pl.ds(sr0, STRIP)])
            return carry

        lax.fori_loop(0, ROWS_T // STRIP, emit_body, 0)

    return body


def _aggregate(table, inc3, gather_row, scatter_row, zeros2d, zdeg, ones):
    return pl.kernel(
        _make_agg_body(gather_row, scatter_row),
        out_type=jax.ShapeDtypeStruct((NC, NPAD, D), jnp.float32),
        mesh=_sc_mesh(),
        scratch_types=[
            pltpu.VMEM((KB, CHUNK), jnp.int32),
            pltpu.VMEM((KB, CHUNK), jnp.int32),
            pltpu.VMEM((CHUNK, D), jnp.float32),
            pltpu.VMEM((CHUNK, D), jnp.float32),
            pltpu.VMEM((CHUNK,), jnp.float32),
            pltpu.VMEM((STRIP, D), jnp.float32),
            pltpu.VMEM((STRIP,), jnp.float32),
            pltpu.SemaphoreType.DMA,
            pltpu.SemaphoreType.DMA,
            pltpu.SemaphoreType.DMA,
            pltpu.VMEM_SHARED((NPAD, D), jnp.float32),
            pltpu.VMEM_SHARED((NPAD,), jnp.float32),
        ],
    )(table, inc3, zeros2d, zdeg, ones)


@jax.jit
def kernel(t, y, incidence, theta, bias):
    del t
    zeros2d = jnp.zeros((ROWS_T, D), jnp.float32)
    zdeg = jnp.zeros((ROWS_T,), jnp.float32)
    ones = jnp.ones((CHUNK,), jnp.float32)

    y_pad = jnp.pad(y, ((0, NPAD - N_NODES), (0, 0)))
    pad_idx = PAD_IDX + (jnp.arange(NNZ_P - NNZ, dtype=jnp.int32)
                         % (NPAD - PAD_IDX))
    inc3 = jnp.concatenate(
        [incidence, jnp.broadcast_to(pad_idx, (2, NNZ_P - NNZ))], axis=1
    ).reshape(2, NCHUNKS_P, CHUNK)

    y_proj = _project(y_pad)
    edge_parts = _aggregate(y_proj, inc3, 0, 1, zeros2d, zdeg, ones)
    edge_feat = _combine(edge_parts)
    node_parts = _aggregate(edge_feat, inc3, 1, 0, zeros2d, zdeg, ones)
    return _final(node_parts, theta, bias)


# speedup vs baseline: 2.6990x; 1.0042x over previous
import jax
import jax.numpy as jnp
from jax import lax
from jax.experimental import pallas as pl
from jax.experimental.pallas import tpu as pltpu
from jax.experimental.pallas import tpu_sc as plsc

N_NODES = 10000
N_EDGES = 10000
NNZ = 320000
D = 128

CHUNK = 128
NC = 2
NS = 16
NW = NC * NS
NPAD = 10240
PAD_IDX = 10000
ROWS_T = NPAD // NS
STRIP = 32

NCHUNKS_P = 2560
NNZ_P = NCHUNKS_P * CHUNK
CPT = NCHUNKS_P // NW
CPT_D = NCHUNKS_P // NS
KB = 8
NBLK = CPT // KB
NBLK_D = CPT_D // KB

_BLK = 2048
_OBLK = 2000


def _project_body(y_ref, o_ref):
    y = y_ref[...]
    ss = jnp.sum(y * y, axis=1, keepdims=True)
    o_ref[...] = y / jnp.maximum(jnp.sqrt(ss), 1e-7)


def _combine_body(p_ref, o_ref):
    o_ref[...] = p_ref[0] + p_ref[1]


def _final_body(p_ref, th_ref, b_ref, o_ref):
    h = p_ref[0] + p_ref[1]
    acc = jnp.dot(h, th_ref[...], preferred_element_type=jnp.float32)
    o_ref[...] = jnp.tanh(acc + b_ref[...])


def _project(y_pad):
    return pl.pallas_call(
        _project_body,
        grid=(NPAD // _BLK,),
        in_specs=[pl.BlockSpec((_BLK, D), lambda i: (i, 0))],
        out_specs=pl.BlockSpec((_BLK, D), lambda i: (i, 0)),
        out_shape=jax.ShapeDtypeStruct((NPAD, D), jnp.float32),
    )(y_pad)


def _combine(parts):
    return pl.pallas_call(
        _combine_body,
        grid=(NPAD // _BLK,),
        in_specs=[pl.BlockSpec((NC, _BLK, D), lambda i: (0, i, 0))],
        out_specs=pl.BlockSpec((_BLK, D), lambda i: (i, 0)),
        out_shape=jax.ShapeDtypeStruct((NPAD, D), jnp.float32),
    )(parts)


def _final(parts, theta, bias):
    return pl.pallas_call(
        _final_body,
        grid=(N_NODES // _OBLK,),
        in_specs=[
            pl.BlockSpec((NC, _OBLK, D), lambda i: (0, i, 0)),
            pl.BlockSpec((D, D), lambda i: (0, 0)),
            pl.BlockSpec((1, D), lambda i: (0, 0)),
        ],
        out_specs=pl.BlockSpec((_OBLK, D), lambda i: (i, 0)),
        out_shape=jax.ShapeDtypeStruct((N_NODES, D), jnp.float32),
    )(parts, theta, bias.reshape(1, D))


def _sc_mesh():
    return plsc.VectorSubcoreMesh(core_axis_name="c", subcore_axis_name="s")


def _make_agg_body(gather_row, scatter_row):
    def body(table_hbm, inc3_hbm, zeros2d_hbm, zdeg_hbm, ones_hbm,
             parts_hbm,
             gidx2, sidx2, rows0, rows1, ones_v, acc_v, degl_v,
             semg0, semg1, sems0, sems1, semd,
             acc_sh, deg_sh):
        cid = lax.axis_index("c")
        sid = lax.axis_index("s")
        wid = sid * NC + cid
        r0 = sid * ROWS_T

        pltpu.sync_copy(ones_hbm, ones_v)

        pltpu.sync_copy(zeros2d_hbm, acc_sh.at[pl.ds(r0, ROWS_T)])
        pltpu.sync_copy(zdeg_hbm, deg_sh.at[pl.ds(r0, ROWS_T)])

        plsc.subcore_barrier()

        c0 = wid * CPT
        rows = (rows0, rows1)
        semgs = (semg0, semg1)
        semss = (sems0, sems1)

        def feat_body(bi, carry):
            cb = c0 + bi * KB
            pltpu.sync_copy(inc3_hbm.at[gather_row, pl.ds(cb, KB)], gidx2)
            pltpu.sync_copy(inc3_hbm.at[scatter_row, pl.ds(cb, KB)], sidx2)
            pend_g = pltpu.async_copy(table_hbm.at[gidx2.at[0]], rows0, semg0)
            pend_s = None
            for j in range(KB):
                b = j % 2
                nb = 1 - b
                pend_g.wait()
                if pend_s is not None:
                    pend_s.wait()
                if j + 1 < KB:
                    pend_g = pltpu.async_copy(
                        table_hbm.at[gidx2.at[j + 1]], rows[nb], semgs[nb])
                pend_s = pltpu.async_copy(
                    rows[b], acc_sh.at[sidx2.at[j]], semss[b], add=True)
            pend_s.wait()
            return carry

        lax.fori_loop(0, NBLK, feat_body, 0)

        c0d = sid * CPT_D

        def deg_body(bi, carry):
            cb = c0d + bi * KB
            pltpu.sync_copy(inc3_hbm.at[scatter_row, pl.ds(cb, KB)], sidx2)
            pend = []
            for j in range(KB):
                pend.append(pltpu.async_copy(
                    ones_v, deg_sh.at[sidx2.at[j]], semd, add=True))
            for p in pend:
                p.wait()
            return carry

        lax.fori_loop(0, NBLK_D, deg_body, 0)

        plsc.subcore_barrier()

        def emit_body(si, carry):
            sr0 = r0 + si * STRIP
            pltpu.sync_copy(acc_sh.at[pl.ds(sr0, STRIP)], acc_v)
            pltpu.sync_copy(deg_sh.at[pl.ds(sr0, STRIP)], degl_v)
            for g in range(STRIP // 16):
                dv = degl_v[pl.ds(16 * g, 16)]
                rec = 1.0 / jnp.maximum(dv, 1.0)
                for k in range(16):
                    r = 16 * g + k
                    s = rec[k]
                    for j in range(D // 16):
                        acc_v[r, pl.ds(16 * j, 16)] = acc_v[r, pl.ds(16 * j, 16)] * s
            pltpu.sync_copy(acc_v, parts_hbm.at[cid, pl.ds(sr0, STRIP)])
            return carry

        lax.fori_loop(0, ROWS_T // STRIP, emit_body, 0)

    return body


def _aggregate(table, inc3, gather_row, scatter_row, zeros2d, zdeg, ones):
    return pl.kernel(
        _make_agg_body(gather_row, scatter_row),
        out_type=jax.ShapeDtypeStruct((NC, NPAD, D), jnp.float32),
        mesh=_sc_mesh(),
        scratch_types=[
            pltpu.VMEM((KB, CHUNK), jnp.int32),
            pltpu.VMEM((KB, CHUNK), jnp.int32),
            pltpu.VMEM((CHUNK, D), jnp.float32),
            pltpu.VMEM((CHUNK, D), jnp.float32),
            pltpu.VMEM((CHUNK,), jnp.float32),
            pltpu.VMEM((STRIP, D), jnp.float32),
            pltpu.VMEM((STRIP,), jnp.float32),
            pltpu.SemaphoreType.DMA,
            pltpu.SemaphoreType.DMA,
            pltpu.SemaphoreType.DMA,
            pltpu.SemaphoreType.DMA,
            pltpu.SemaphoreType.DMA,
            pltpu.VMEM_SHARED((NPAD, D), jnp.float32),
            pltpu.VMEM_SHARED((NPAD,), jnp.float32),
        ],
    )(table, inc3, zeros2d, zdeg, ones)


@jax.jit
def kernel(t, y, incidence, theta, bias):
    del t
    zeros2d = jnp.zeros((ROWS_T, D), jnp.float32)
    zdeg = jnp.zeros((ROWS_T,), jnp.float32)
    ones = jnp.ones((CHUNK,), jnp.float32)

    y_pad = jnp.pad(y, ((0, NPAD - N_NODES), (0, 0)))
    pad_idx = PAD_IDX + (jnp.arange(NNZ_P - NNZ, dtype=jnp.int32)
                         % (NPAD - PAD_IDX))
    inc3 = jnp.concatenate(
        [incidence, jnp.broadcast_to(pad_idx, (2, NNZ_P - NNZ))], axis=1
    ).reshape(2, NCHUNKS_P, CHUNK)

    y_proj = _project(y_pad)
    edge_parts = _aggregate(y_proj, inc3, 0, 1, zeros2d, zdeg, ones)
    edge_feat = _combine(edge_parts)
    node_parts = _aggregate(edge_feat, inc3, 1, 0, zeros2d, zdeg, ones)
    return _final(node_parts, theta, bias)


# speedup vs baseline: 2.9023x; 1.0753x over previous
import jax
import jax.numpy as jnp
from jax import lax
from jax.experimental import pallas as pl
from jax.experimental.pallas import tpu as pltpu
from jax.experimental.pallas import tpu_sc as plsc

N_NODES = 10000
N_EDGES = 10000
NNZ = 320000
D = 128

CHUNK = 128
NC = 2
NS = 16
NW = NC * NS
NPAD = 10240
PAD_IDX = 10000
ROWS_T = NPAD // NS
STRIP = 32

NCHUNKS_P = 2560
NNZ_P = NCHUNKS_P * CHUNK
CPT = NCHUNKS_P // NW
CPT_D = NCHUNKS_P // NS
KB = 8
NBLK = CPT // KB
NBLK_D = CPT_D // KB

_BLK = 2048
_OBLK = 2000


def _project_body(y_ref, o_ref):
    y = y_ref[...]
    ss = jnp.sum(y * y, axis=1, keepdims=True)
    o_ref[...] = y / jnp.maximum(jnp.sqrt(ss), 1e-7)


def _combine_body(p_ref, o_ref):
    o_ref[...] = p_ref[0] + p_ref[1]


def _final_body(p_ref, th_ref, b_ref, o_ref):
    h = p_ref[0] + p_ref[1]
    acc = jnp.dot(h, th_ref[...], preferred_element_type=jnp.float32)
    o_ref[...] = jnp.tanh(acc + b_ref[...])


def _project(y_pad):
    return pl.pallas_call(
        _project_body,
        grid=(NPAD // _BLK,),
        in_specs=[pl.BlockSpec((_BLK, D), lambda i: (i, 0))],
        out_specs=pl.BlockSpec((_BLK, D), lambda i: (i, 0)),
        out_shape=jax.ShapeDtypeStruct((NPAD, D), jnp.float32),
    )(y_pad)


def _combine(parts):
    return pl.pallas_call(
        _combine_body,
        grid=(NPAD // _BLK,),
        in_specs=[pl.BlockSpec((NC, _BLK, D), lambda i: (0, i, 0))],
        out_specs=pl.BlockSpec((_BLK, D), lambda i: (i, 0)),
        out_shape=jax.ShapeDtypeStruct((NPAD, D), jnp.float32),
    )(parts)


def _final(parts, theta, bias):
    return pl.pallas_call(
        _final_body,
        grid=(N_NODES // _OBLK,),
        in_specs=[
            pl.BlockSpec((NC, _OBLK, D), lambda i: (0, i, 0)),
            pl.BlockSpec((D, D), lambda i: (0, 0)),
            pl.BlockSpec((1, D), lambda i: (0, 0)),
        ],
        out_specs=pl.BlockSpec((_OBLK, D), lambda i: (i, 0)),
        out_shape=jax.ShapeDtypeStruct((N_NODES, D), jnp.float32),
    )(parts, theta, bias.reshape(1, D))


def _sc_mesh():
    return plsc.VectorSubcoreMesh(core_axis_name="c", subcore_axis_name="s")


def _make_agg_body(gather_row, scatter_row):
    def body(table_hbm, inc3_hbm, zeros2d_hbm, zdeg_hbm, ones_hbm,
             parts_hbm,
             gidx2, sidx2, rows0, rows1, ones_v, acc_v, degl_v,
             semg0, semg1, sems0, sems1, semd,
             acc_sh, deg_sh):
        cid = lax.axis_index("c")
        sid = lax.axis_index("s")
        wid = sid * NC + cid
        r0 = sid * ROWS_T

        pltpu.sync_copy(ones_hbm, ones_v)

        pltpu.sync_copy(zeros2d_hbm, acc_sh.at[pl.ds(r0, ROWS_T)])
        pltpu.sync_copy(zdeg_hbm, deg_sh.at[pl.ds(r0, ROWS_T)])

        plsc.subcore_barrier()

        c0 = wid * CPT
        rows = (rows0, rows1)
        semgs = (semg0, semg1)
        semss = (sems0, sems1)

        def feat_body(bi, carry):
            cb = c0 + bi * KB
            pltpu.sync_copy(inc3_hbm.at[gather_row, pl.ds(cb, KB)], gidx2)
            pltpu.sync_copy(inc3_hbm.at[scatter_row, pl.ds(cb, KB)], sidx2)
            pend_g = [None, None]
            pend_g[0] = pltpu.async_copy(table_hbm.at[gidx2.at[0]], rows0, semg0)
            pend_s = None
            for j in range(KB):
                b = j % 2
                nb = 1 - b
                if pend_s is not None:
                    pend_s.wait()
                if j + 1 < KB:
                    pend_g[nb] = pltpu.async_copy(
                        table_hbm.at[gidx2.at[j + 1]], rows[nb], semgs[nb])
                pend_g[b].wait()
                pend_s = pltpu.async_copy(
                    rows[b], acc_sh.at[sidx2.at[j]], semss[b], add=True)
            pend_s.wait()
            return carry

        lax.fori_loop(0, NBLK, feat_body, 0)

        c0d = sid * CPT_D

        def deg_body(bi, carry):
            cb = c0d + bi * KB
            pltpu.sync_copy(inc3_hbm.at[scatter_row, pl.ds(cb, KB)], sidx2)
            pend = []
            for j in range(KB):
                pend.append(pltpu.async_copy(
                    ones_v, deg_sh.at[sidx2.at[j]], semd, add=True))
            for p in pend:
                p.wait()
            return carry

        lax.fori_loop(0, NBLK_D, deg_body, 0)

        plsc.subcore_barrier()

        def emit_body(si, carry):
            sr0 = r0 + si * STRIP
            pltpu.sync_copy(acc_sh.at[pl.ds(sr0, STRIP)], acc_v)
            pltpu.sync_copy(deg_sh.at[pl.ds(sr0, STRIP)], degl_v)
            for g in range(STRIP // 16):
                dv = degl_v[pl.ds(16 * g, 16)]
                rec = 1.0 / jnp.maximum(dv, 1.0)
                for k in range(16):
                    r = 16 * g + k
                    s = rec[k]
                    for j in range(D // 16):
                        acc_v[r, pl.ds(16 * j, 16)] = acc_v[r, pl.ds(16 * j, 16)] * s
            pltpu.sync_copy(acc_v, parts_hbm.at[cid, pl.ds(sr0, STRIP)])
            return carry

        lax.fori_loop(0, ROWS_T // STRIP, emit_body, 0)

    return body


def _aggregate(table, inc3, gather_row, scatter_row, zeros2d, zdeg, ones):
    return pl.kernel(
        _make_agg_body(gather_row, scatter_row),
        out_type=jax.ShapeDtypeStruct((NC, NPAD, D), jnp.float32),
        mesh=_sc_mesh(),
        scratch_types=[
            pltpu.VMEM((KB, CHUNK), jnp.int32),
            pltpu.VMEM((KB, CHUNK), jnp.int32),
            pltpu.VMEM((CHUNK, D), jnp.float32),
            pltpu.VMEM((CHUNK, D), jnp.float32),
            pltpu.VMEM((CHUNK,), jnp.float32),
            pltpu.VMEM((STRIP, D), jnp.float32),
            pltpu.VMEM((STRIP,), jnp.float32),
            pltpu.SemaphoreType.DMA,
            pltpu.SemaphoreType.DMA,
            pltpu.SemaphoreType.DMA,
            pltpu.SemaphoreType.DMA,
            pltpu.SemaphoreType.DMA,
            pltpu.VMEM_SHARED((NPAD, D), jnp.float32),
            pltpu.VMEM_SHARED((NPAD,), jnp.float32),
        ],
    )(table, inc3, zeros2d, zdeg, ones)


@jax.jit
def kernel(t, y, incidence, theta, bias):
    del t
    zeros2d = jnp.zeros((ROWS_T, D), jnp.float32)
    zdeg = jnp.zeros((ROWS_T,), jnp.float32)
    ones = jnp.ones((CHUNK,), jnp.float32)

    y_pad = jnp.pad(y, ((0, NPAD - N_NODES), (0, 0)))
    pad_idx = PAD_IDX + (jnp.arange(NNZ_P - NNZ, dtype=jnp.int32)
                         % (NPAD - PAD_IDX))
    inc3 = jnp.concatenate(
        [incidence, jnp.broadcast_to(pad_idx, (2, NNZ_P - NNZ))], axis=1
    ).reshape(2, NCHUNKS_P, CHUNK)

    y_proj = _project(y_pad)
    edge_parts = _aggregate(y_proj, inc3, 0, 1, zeros2d, zdeg, ones)
    edge_feat = _combine(edge_parts)
    node_parts = _aggregate(edge_feat, inc3, 1, 0, zeros2d, zdeg, ones)
    return _final(node_parts, theta, bias)


# speedup vs baseline: 3.2104x; 1.1062x over previous
import jax
import jax.numpy as jnp
from jax import lax
from jax.experimental import pallas as pl
from jax.experimental.pallas import tpu as pltpu
from jax.experimental.pallas import tpu_sc as plsc

N_NODES = 10000
N_EDGES = 10000
NNZ = 320000
D = 128

CHUNK = 128
NC = 2
NS = 16
NW = NC * NS
NPAD = 10240
PAD_IDX = 10000
ROWS_T = NPAD // NS
STRIP = 32

NCHUNKS_P = 2560
NNZ_P = NCHUNKS_P * CHUNK
CPT = NCHUNKS_P // NW
CPT_D = NCHUNKS_P // NS
KB = 8
NBLK = CPT // KB
NBLK_D = CPT_D // KB
NCHUNKS_ALLOC = NCHUNKS_P + 2 * KB

_BLK = 2048
_OBLK = 2000


def _project_body(y_ref, o_ref):
    y = y_ref[...]
    ss = jnp.sum(y * y, axis=1, keepdims=True)
    o_ref[...] = y / jnp.maximum(jnp.sqrt(ss), 1e-7)


def _combine_body(p_ref, o_ref):
    o_ref[...] = p_ref[0] + p_ref[1]


def _final_body(p_ref, th_ref, b_ref, o_ref):
    h = p_ref[0] + p_ref[1]
    acc = jnp.dot(h, th_ref[...], preferred_element_type=jnp.float32)
    o_ref[...] = jnp.tanh(acc + b_ref[...])


def _project(y_pad):
    return pl.pallas_call(
        _project_body,
        grid=(NPAD // _BLK,),
        in_specs=[pl.BlockSpec((_BLK, D), lambda i: (i, 0))],
        out_specs=pl.BlockSpec((_BLK, D), lambda i: (i, 0)),
        out_shape=jax.ShapeDtypeStruct((NPAD, D), jnp.float32),
    )(y_pad)


def _combine(parts):
    return pl.pallas_call(
        _combine_body,
        grid=(NPAD // _BLK,),
        in_specs=[pl.BlockSpec((NC, _BLK, D), lambda i: (0, i, 0))],
        out_specs=pl.BlockSpec((_BLK, D), lambda i: (i, 0)),
        out_shape=jax.ShapeDtypeStruct((NPAD, D), jnp.float32),
    )(parts)


def _final(parts, theta, bias):
    return pl.pallas_call(
        _final_body,
        grid=(N_NODES // _OBLK,),
        in_specs=[
            pl.BlockSpec((NC, _OBLK, D), lambda i: (0, i, 0)),
            pl.BlockSpec((D, D), lambda i: (0, 0)),
            pl.BlockSpec((1, D), lambda i: (0, 0)),
        ],
        out_specs=pl.BlockSpec((_OBLK, D), lambda i: (i, 0)),
        out_shape=jax.ShapeDtypeStruct((N_NODES, D), jnp.float32),
    )(parts, theta, bias.reshape(1, D))


def _sc_mesh():
    return plsc.VectorSubcoreMesh(core_axis_name="c", subcore_axis_name="s")


def _make_agg_body(gather_row, scatter_row):
    def body(table_hbm, inc3_hbm, zeros2d_hbm, zdeg_hbm, ones_hbm,
             parts_hbm,
             gidxa, sidxa, gidxb, sidxb, rows0, rows1, ones_v, acc_v, degl_v,
             semg0, semg1, sems0, sems1, semd, semi,
             acc_sh, deg_sh):
        cid = lax.axis_index("c")
        sid = lax.axis_index("s")
        wid = sid * NC + cid
        r0 = sid * ROWS_T

        pltpu.sync_copy(ones_hbm, ones_v)

        pltpu.sync_copy(zeros2d_hbm, acc_sh.at[pl.ds(r0, ROWS_T)])
        pltpu.sync_copy(zdeg_hbm, deg_sh.at[pl.ds(r0, ROWS_T)])

        plsc.subcore_barrier()

        c0 = wid * CPT
        rows = (rows0, rows1)
        semgs = (semg0, semg1)
        semss = (sems0, sems1)

        def idx_issue(cb, gbuf, sbuf):
            pltpu.async_copy(inc3_hbm.at[gather_row, pl.ds(cb, KB)], gbuf, semi)
            pltpu.async_copy(inc3_hbm.at[scatter_row, pl.ds(cb, KB)], sbuf, semi)

        def idx_wait(cb, gbuf, sbuf):
            pltpu.make_async_copy(
                inc3_hbm.at[gather_row, pl.ds(cb, KB)], gbuf, semi).wait()
            pltpu.make_async_copy(
                inc3_hbm.at[scatter_row, pl.ds(cb, KB)], sbuf, semi).wait()

        idx_issue(c0, gidxa, sidxa)

        def feat_body(q, carry):
            cba = c0 + q * 2 * KB
            cbb = cba + KB
            idx_wait(cba, gidxa, sidxa)
            idx_issue(cbb, gidxb, sidxb)

            def gref(j):
                return gidxa.at[j] if j < KB else gidxb.at[j - KB]

            def sref(j):
                return sidxa.at[j] if j < KB else sidxb.at[j - KB]

            pend_g = [None, None]
            pend_g[0] = pltpu.async_copy(table_hbm.at[gref(0)], rows0, semg0)
            pend_s = None
            for j in range(2 * KB):
                b = j % 2
                nb = 1 - b
                if j == KB - 1:
                    idx_wait(cbb, gidxb, sidxb)
                if j == KB + 1:
                    idx_issue(cba + 2 * KB, gidxa, sidxa)
                if pend_s is not None:
                    pend_s.wait()
                if j + 1 < 2 * KB:
                    pend_g[nb] = pltpu.async_copy(
                        table_hbm.at[gref(j + 1)], rows[nb], semgs[nb])
                pend_g[b].wait()
                pend_s = pltpu.async_copy(
                    rows[b], acc_sh.at[sref(j)], semss[b], add=True)
            pend_s.wait()
            return carry

        lax.fori_loop(0, NBLK // 2, feat_body, 0)
        idx_wait(c0, gidxa, sidxa)

        c0d = sid * CPT_D

        def deg_body(bi, carry):
            cb = c0d + bi * KB
            pltpu.sync_copy(inc3_hbm.at[scatter_row, pl.ds(cb, KB)], sidxa)
            pend = []
            for j in range(KB):
                pend.append(pltpu.async_copy(
                    ones_v, deg_sh.at[sidxa.at[j]], semd, add=True))
            for p in pend:
                p.wait()
            return carry

        lax.fori_loop(0, NBLK_D, deg_body, 0)

        plsc.subcore_barrier()

        def emit_body(si, carry):
            sr0 = r0 + si * STRIP
            pltpu.sync_copy(acc_sh.at[pl.ds(sr0, STRIP)], acc_v)
            pltpu.sync_copy(deg_sh.at[pl.ds(sr0, STRIP)], degl_v)
            for g in range(STRIP // 16):
                dv = degl_v[pl.ds(16 * g, 16)]
                rec = 1.0 / jnp.maximum(dv, 1.0)
                for k in range(16):
                    r = 16 * g + k
                    s = rec[k]
                    for j in range(D // 16):
                        acc_v[r, pl.ds(16 * j, 16)] = acc_v[r, pl.ds(16 * j, 16)] * s
            pltpu.sync_copy(acc_v, parts_hbm.at[cid, pl.ds(sr0, STRIP)])
            return carry

        lax.fori_loop(0, ROWS_T // STRIP, emit_body, 0)

    return body


def _aggregate(table, inc3, gather_row, scatter_row, zeros2d, zdeg, ones):
    return pl.kernel(
        _make_agg_body(gather_row, scatter_row),
        out_type=jax.ShapeDtypeStruct((NC, NPAD, D), jnp.float32),
        mesh=_sc_mesh(),
        scratch_types=[
            pltpu.VMEM((KB, CHUNK), jnp.int32),
            pltpu.VMEM((KB, CHUNK), jnp.int32),
            pltpu.VMEM((KB, CHUNK), jnp.int32),
            pltpu.VMEM((KB, CHUNK), jnp.int32),
            pltpu.VMEM((CHUNK, D), jnp.float32),
            pltpu.VMEM((CHUNK, D), jnp.float32),
            pltpu.VMEM((CHUNK,), jnp.float32),
            pltpu.VMEM((STRIP, D), jnp.float32),
            pltpu.VMEM((STRIP,), jnp.float32),
            pltpu.SemaphoreType.DMA,
            pltpu.SemaphoreType.DMA,
            pltpu.SemaphoreType.DMA,
            pltpu.SemaphoreType.DMA,
            pltpu.SemaphoreType.DMA,
            pltpu.SemaphoreType.DMA,
            pltpu.VMEM_SHARED((NPAD, D), jnp.float32),
            pltpu.VMEM_SHARED((NPAD,), jnp.float32),
        ],
    )(table, inc3, zeros2d, zdeg, ones)


@jax.jit
def kernel(t, y, incidence, theta, bias):
    del t
    zeros2d = jnp.zeros((ROWS_T, D), jnp.float32)
    zdeg = jnp.zeros((ROWS_T,), jnp.float32)
    ones = jnp.ones((CHUNK,), jnp.float32)

    y_pad = jnp.pad(y, ((0, NPAD - N_NODES), (0, 0)))
    nnz_alloc = NCHUNKS_ALLOC * CHUNK
    pad_idx = PAD_IDX + (jnp.arange(nnz_alloc - NNZ, dtype=jnp.int32)
                         % (NPAD - PAD_IDX))
    inc3 = jnp.concatenate(
        [incidence, jnp.broadcast_to(pad_idx, (2, nnz_alloc - NNZ))], axis=1
    ).reshape(2, NCHUNKS_ALLOC, CHUNK)

    y_proj = _project(y_pad)
    edge_parts = _aggregate(y_proj, inc3, 0, 1, zeros2d, zdeg, ones)
    edge_feat = _combine(edge_parts)
    node_parts = _aggregate(edge_feat, inc3, 1, 0, zeros2d, zdeg, ones)
    return _final(node_parts, theta, bias)
